# baseline (speedup 1.0000x reference)
import functools

import jax
import jax.numpy as jnp
from jax import lax
from jax.experimental import pallas as pl
from jax.experimental.pallas import tpu as pltpu
from jax.experimental.pallas import tpu_sc as plsc

H, W = 800, 800
TANFOVX, TANFOVY = 0.5, 0.5
SCALE_MOD = 1.0
_N = 200000
_NPAD = 212992
_NROWS = _NPAD // 128
_BR = 208
_HW = H * W
_PROWS = _HW // 128
_PBR = 1000
_NSUB = 16
_CHUNK = _NPAD // _NSUB
_KJ = _CHUNK // 128
_STRIPE = _HW // _NSUB
_ZB = 8000
_FIRE = 8


def _bfr(v):
    return v.astype(jnp.bfloat16).astype(jnp.float32)


def _phase1_body(params_ref, pk_ref, vals_ref, ints_ref):
    def P(i):
        return params_ref[0, i]
    vm = [[P(r * 4 + c) for c in range(4)] for r in range(4)]
    pm = [[P(16 + r * 4 + c) for c in range(4)] for r in range(4)]
    x = _bfr(pk_ref[0])
    y = _bfr(pk_ref[1])
    z = _bfr(pk_ref[2])
    tx = x * vm[0][0] + y * vm[1][0] + z * vm[2][0] + vm[3][0]
    ty = x * vm[0][1] + y * vm[1][1] + z * vm[2][1] + vm[3][1]
    tz = x * vm[0][2] + y * vm[1][2] + z * vm[2][2] + vm[3][2]
    hx = x * pm[0][0] + y * pm[1][0] + z * pm[2][0] + pm[3][0]
    hy = x * pm[0][1] + y * pm[1][1] + z * pm[2][1] + pm[3][1]
    hw = x * pm[0][3] + y * pm[1][3] + z * pm[2][3] + pm[3][3]
    pw = 1.0 / (hw + 1e-7)
    px = ((hx * pw + 1.0) * W - 1.0) * 0.5
    py = ((hy * pw + 1.0) * H - 1.0) * 0.5
    qr = pk_ref[10]
    qx = pk_ref[11]
    qy = pk_ref[12]
    qz = pk_ref[13]
    den = jnp.sqrt(qr * qr + qx * qx + qy * qy + qz * qz) + 1e-8
    r = qr / den
    xq = qx / den
    yq = qy / den
    zq = qz / den
    R = [[1.0 - 2.0 * (yq * yq + zq * zq), 2.0 * (xq * yq - r * zq), 2.0 * (xq * zq + r * yq)],
         [2.0 * (xq * yq + r * zq), 1.0 - 2.0 * (xq * xq + zq * zq), 2.0 * (yq * zq - r * xq)],
         [2.0 * (xq * zq - r * yq), 2.0 * (yq * zq + r * xq), 1.0 - 2.0 * (xq * xq + yq * yq)]]
    s = [pk_ref[7] * SCALE_MOD, pk_ref[8] * SCALE_MOD, pk_ref[9] * SCALE_MOD]
    M = [[_bfr(R[i][k] * s[k]) for k in range(3)] for i in range(3)]
    cov = [[M[i][0] * M[j][0] + M[i][1] * M[j][1] + M[i][2] * M[j][2]
            for j in range(3)] for i in range(3)]
    Wr = [[vm[j][i] for j in range(3)] for i in range(3)]
    covb = [[_bfr(cov[j][k]) for k in range(3)] for j in range(3)]
    T = [[Wr[i][0] * covb[0][k] + Wr[i][1] * covb[1][k] + Wr[i][2] * covb[2][k]
          for k in range(3)] for i in range(3)]
    V = [[_bfr(T[i][0]) * Wr[l][0] + _bfr(T[i][1]) * Wr[l][1] + _bfr(T[i][2]) * Wr[l][2]
          for l in range(3)] for i in range(3)]
    fx = W / (2.0 * TANFOVX)
    fy = H / (2.0 * TANFOVY)
    tzc = jnp.maximum(tz, 1e-3)
    a = fx / tzc
    b = fy / tzc
    c1 = -fx * tx / (tzc * tzc)
    c2 = -fy * ty / (tzc * tzc)
    c00 = a * a * V[0][0] + 2.0 * a * c1 * V[0][2] + c1 * c1 * V[2][2] + 0.3
    c01 = a * b * V[0][1] + a * c2 * V[0][2] + b * c1 * V[1][2] + c1 * c2 * V[2][2]
    c11 = b * b * V[1][1] + 2.0 * b * c2 * V[1][2] + c2 * c2 * V[2][2] + 0.3
    det = c00 * c11 - c01 * c01
    mid = 0.5 * (c00 + c11)
    lam = mid + jnp.sqrt(jnp.maximum(mid * mid - det, 0.1))
    radius_f = jnp.ceil(3.0 * jnp.sqrt(jnp.maximum(lam, 0.0)))
    maskb = ((tz > 0.2) & (px >= 0.0) & (px <= W - 1.0)
             & (py >= 0.0) & (py <= H - 1.0) & (det > 0.0))
    mask = maskb.astype(jnp.float32)
    rad_i = jnp.where(maskb, radius_f, 0.0).astype(jnp.int32)
    ix = jnp.clip(jnp.round(px), 0, W - 1).astype(jnp.int32)
    iy = jnp.clip(jnp.round(py), 0, H - 1).astype(jnp.int32)
    alpha = jnp.clip(pk_ref[6], 0.0, 0.99) * mask
    vals_ref[0, :, :] = alpha * pk_ref[3]
    vals_ref[1, :, :] = alpha * pk_ref[4]
    vals_ref[2, :, :] = alpha * pk_ref[5]
    vals_ref[3, :, :] = alpha
    vals_ref[4, :, :] = alpha * tz
    ints_ref[0, :, :] = iy * W + ix
    ints_ref[1, :, :] = rad_i
    tw = 2 * rad_i + 1
    ints_ref[2, :, :] = jnp.where(maskb, tw * tw, 0)


def _phase1(packed, params, interpret=False):
    grid = _NROWS // _BR
    return pl.pallas_call(
        _phase1_body,
        grid=(grid,),
        in_specs=[
            pl.BlockSpec((1, 128), lambda i: (0, 0)),
            pl.BlockSpec((14, _BR, 128), lambda i: (0, i, 0)),
        ],
        out_specs=[
            pl.BlockSpec((5, _BR, 128), lambda i: (0, i, 0)),
            pl.BlockSpec((3, _BR, 128), lambda i: (0, i, 0)),
        ],
        out_shape=[
            jax.ShapeDtypeStruct((5, _NROWS, 128), jnp.float32),
            jax.ShapeDtypeStruct((3, _NROWS, 128), jnp.int32),
        ],
        interpret=interpret,
    )(params, packed)


def _sc_scatter_body(flat_hbm, vals_hbm, out_hbm,
                     idx_v, val_v, zbuf, obuf, acc0, acc1, sem):
    c = lax.axis_index("c")
    s = lax.axis_index("s")
    accs = [acc0, acc1]

    def zfill(i, _):
        zbuf[pl.ds(i * 16, 16)] = jnp.zeros((16,), jnp.float32)
        return 0
    lax.fori_loop(0, _ZB // 16, zfill, 0)
    pltpu.sync_copy(flat_hbm.at[s], idx_v)

    def scatter(k, chg, glo, ghi):
        def group(g, _):
            for b in range(_FIRE):
                j = g * _FIRE + b
                pltpu.async_copy(val_v.at[j], accs[k].at[idx_v.at[j]],
                                 sem, add=True)
            for b in range(_FIRE):
                j = g * _FIRE + b
                pltpu.make_async_copy(val_v.at[j], accs[k].at[idx_v.at[j]],
                                      sem).wait()
            return 0
        pltpu.sync_copy(vals_hbm.at[chg, s], val_v)
        lax.fori_loop(glo, ghi, group, 0)

    def zero(k):
        for t in range(_STRIPE // _ZB):
            pltpu.sync_copy(zbuf, accs[k].at[pl.ds(s * _STRIPE + t * _ZB, _ZB)])

    def writeout(k, chg):
        for t in range(_STRIPE // _ZB):
            sl = pl.ds(s * _STRIPE + t * _ZB, _ZB)
            pltpu.sync_copy(accs[k].at[sl], obuf)
            off = chg * _HW + s * _STRIPE + t * _ZB
            pltpu.sync_copy(obuf, out_hbm.at[pl.ds(off, _ZB)])

    ngrp = _KJ // _FIRE
    zero(0)
    zero(1)
    plsc.subcore_barrier()
    scatter(0, 2 * c + 0, 0, ngrp)
    scatter(1, 2 * c + 1, 0, ngrp)
    plsc.subcore_barrier()
    writeout(0, 2 * c + 0)
    writeout(1, 2 * c + 1)
    zero(0)
    plsc.subcore_barrier()
    scatter(0, 4, 7 * c, 7 + 6 * c)
    plsc.subcore_barrier()
    writeout(0, 4 + c)


def _sc_scatter(flat3, vals4, interpret=False):
    mesh = plsc.VectorSubcoreMesh(core_axis_name="c", subcore_axis_name="s",
                                  num_cores=2, num_subcores=_NSUB)
    return pl.kernel(
        _sc_scatter_body,
        out_type=jax.ShapeDtypeStruct((6 * _HW,), jnp.float32),
        mesh=mesh,
        scratch_types=[
            pltpu.VMEM((_KJ, 128), jnp.int32),
            pltpu.VMEM((_KJ, 128), jnp.float32),
            pltpu.VMEM((_ZB,), jnp.float32),
            pltpu.VMEM((_ZB,), jnp.float32),
            pltpu.VMEM_SHARED((_HW,), jnp.float32),
            pltpu.VMEM_SHARED((_HW,), jnp.float32),
            pltpu.SemaphoreType.DMA,
        ],
        interpret=interpret,
    )(flat3, vals4)


def _phase3_body(bgp_ref, acc_ref, col_ref, opac_ref, dep_ref):
    acca = acc_ref[3]
    accd = acc_ref[4] + acc_ref[5]
    Tt = jnp.clip(1.0 - acca, 0.0, 1.0)
    col_ref[0, :, :] = acc_ref[0] + Tt * bgp_ref[0, 0]
    col_ref[1, :, :] = acc_ref[1] + Tt * bgp_ref[0, 1]
    col_ref[2, :, :] = acc_ref[2] + Tt * bgp_ref[0, 2]
    opac_ref[:, :] = jnp.clip(acca, 0.0, 1.0)
    dep_ref[:, :] = accd / (acca + 1e-6)


def _phase3(accs, bgp, interpret=False):
    grid = _PROWS // _PBR
    return pl.pallas_call(
        _phase3_body,
        grid=(grid,),
        in_specs=[
            pl.BlockSpec((1, 128), lambda i: (0, 0)),
            pl.BlockSpec((6, _PBR, 128), lambda i: (0, i, 0)),
        ],
        out_specs=[
            pl.BlockSpec((3, _PBR, 128), lambda i: (0, i, 0)),
            pl.BlockSpec((_PBR, 128), lambda i: (i, 0)),
            pl.BlockSpec((_PBR, 128), lambda i: (i, 0)),
        ],
        out_shape=[
            jax.ShapeDtypeStruct((3, _PROWS, 128), jnp.float32),
            jax.ShapeDtypeStruct((_PROWS, 128), jnp.float32),
            jax.ShapeDtypeStruct((_PROWS, 128), jnp.float32),
        ],
        interpret=interpret,
    )(bgp, accs)


def kernel(means3D, means2D, sh, colors_precomp, opacities, scales, rotations,
           theta, rho, viewmatrix, projmatrix, campos, bg):
    pad = _NPAD - _N
    packed = jnp.concatenate(
        [means3D, colors_precomp, opacities, scales, rotations], axis=1)
    packed = jnp.pad(packed, ((0, pad), (0, 0))).T.reshape(14, _NROWS, 128)
    params = jnp.zeros((1, 128), jnp.float32)
    params = params.at[0, :16].set(viewmatrix.reshape(-1))
    params = params.at[0, 16:32].set(projmatrix.reshape(-1))
    params = params.astype(jnp.bfloat16).astype(jnp.float32)
    vals, ints = _phase1(packed, params)
    accs = _sc_scatter(ints[0].reshape(_NSUB, _KJ, 128),
                       vals.reshape(5, _NSUB, _KJ, 128))
    bgp = jnp.zeros((1, 128), jnp.float32).at[0, :3].set(bg)
    colf, opacf, depf = _phase3(accs.reshape(6, _PROWS, 128), bgp)
    radnt = ints[1:].reshape(2, _NPAD)
    return (colf.reshape(3, H, W), radnt[0, :_N], depf.reshape(1, H, W),
            opacf.reshape(1, H, W), radnt[1, :_N])


# speedup vs baseline: 1.1046x; 1.1046x over previous
import functools

import jax
import jax.numpy as jnp
from jax import lax
from jax.experimental import pallas as pl
from jax.experimental.pallas import tpu as pltpu
from jax.experimental.pallas import tpu_sc as plsc

H, W = 800, 800
TANFOVX, TANFOVY = 0.5, 0.5
SCALE_MOD = 1.0
_N = 200000
_NPAD = 204800
_NROWS = _NPAD // 128
_BR = 160
_HW = H * W
_PROWS = _HW // 128
_PBR = 1000
_NSUB = 16
_CHUNK = _NPAD // _NSUB
_KJ = _CHUNK // 128
_STRIPE = _HW // _NSUB
_ZB = 8000
_FIRE = 10


def _bfr(v):
    return v.astype(jnp.bfloat16).astype(jnp.float32)


def _phase1_body(params_ref, pk_ref, vals_ref, ints_ref):
    def P(i):
        return params_ref[0, i]
    vm = [[P(r * 4 + c) for c in range(4)] for r in range(4)]
    pm = [[P(16 + r * 4 + c) for c in range(4)] for r in range(4)]
    x = _bfr(pk_ref[0])
    y = _bfr(pk_ref[1])
    z = _bfr(pk_ref[2])
    tx = x * vm[0][0] + y * vm[1][0] + z * vm[2][0] + vm[3][0]
    ty = x * vm[0][1] + y * vm[1][1] + z * vm[2][1] + vm[3][1]
    tz = x * vm[0][2] + y * vm[1][2] + z * vm[2][2] + vm[3][2]
    hx = x * pm[0][0] + y * pm[1][0] + z * pm[2][0] + pm[3][0]
    hy = x * pm[0][1] + y * pm[1][1] + z * pm[2][1] + pm[3][1]
    hw = x * pm[0][3] + y * pm[1][3] + z * pm[2][3] + pm[3][3]
    pw = 1.0 / (hw + 1e-7)
    px = ((hx * pw + 1.0) * W - 1.0) * 0.5
    py = ((hy * pw + 1.0) * H - 1.0) * 0.5
    qr = pk_ref[10]
    qx = pk_ref[11]
    qy = pk_ref[12]
    qz = pk_ref[13]
    den = jnp.sqrt(qr * qr + qx * qx + qy * qy + qz * qz) + 1e-8
    r = qr / den
    xq = qx / den
    yq = qy / den
    zq = qz / den
    R = [[1.0 - 2.0 * (yq * yq + zq * zq), 2.0 * (xq * yq - r * zq), 2.0 * (xq * zq + r * yq)],
         [2.0 * (xq * yq + r * zq), 1.0 - 2.0 * (xq * xq + zq * zq), 2.0 * (yq * zq - r * xq)],
         [2.0 * (xq * zq - r * yq), 2.0 * (yq * zq + r * xq), 1.0 - 2.0 * (xq * xq + yq * yq)]]
    s = [pk_ref[7] * SCALE_MOD, pk_ref[8] * SCALE_MOD, pk_ref[9] * SCALE_MOD]
    M = [[_bfr(R[i][k] * s[k]) for k in range(3)] for i in range(3)]
    cov = [[M[i][0] * M[j][0] + M[i][1] * M[j][1] + M[i][2] * M[j][2]
            for j in range(3)] for i in range(3)]
    Wr = [[vm[j][i] for j in range(3)] for i in range(3)]
    covb = [[_bfr(cov[j][k]) for k in range(3)] for j in range(3)]
    T = [[Wr[i][0] * covb[0][k] + Wr[i][1] * covb[1][k] + Wr[i][2] * covb[2][k]
          for k in range(3)] for i in range(3)]
    V = [[_bfr(T[i][0]) * Wr[l][0] + _bfr(T[i][1]) * Wr[l][1] + _bfr(T[i][2]) * Wr[l][2]
          for l in range(3)] for i in range(3)]
    fx = W / (2.0 * TANFOVX)
    fy = H / (2.0 * TANFOVY)
    tzc = jnp.maximum(tz, 1e-3)
    a = fx / tzc
    b = fy / tzc
    c1 = -fx * tx / (tzc * tzc)
    c2 = -fy * ty / (tzc * tzc)
    c00 = a * a * V[0][0] + 2.0 * a * c1 * V[0][2] + c1 * c1 * V[2][2] + 0.3
    c01 = a * b * V[0][1] + a * c2 * V[0][2] + b * c1 * V[1][2] + c1 * c2 * V[2][2]
    c11 = b * b * V[1][1] + 2.0 * b * c2 * V[1][2] + c2 * c2 * V[2][2] + 0.3
    det = c00 * c11 - c01 * c01
    mid = 0.5 * (c00 + c11)
    lam = mid + jnp.sqrt(jnp.maximum(mid * mid - det, 0.1))
    radius_f = jnp.ceil(3.0 * jnp.sqrt(jnp.maximum(lam, 0.0)))
    maskb = ((tz > 0.2) & (px >= 0.0) & (px <= W - 1.0)
             & (py >= 0.0) & (py <= H - 1.0) & (det > 0.0))
    mask = maskb.astype(jnp.float32)
    rad_i = jnp.where(maskb, radius_f, 0.0).astype(jnp.int32)
    ix = jnp.clip(jnp.round(px), 0, W - 1).astype(jnp.int32)
    iy = jnp.clip(jnp.round(py), 0, H - 1).astype(jnp.int32)
    alpha = jnp.clip(pk_ref[6], 0.0, 0.99) * mask
    vals_ref[0, :, :] = alpha * pk_ref[3]
    vals_ref[1, :, :] = alpha * pk_ref[4]
    vals_ref[2, :, :] = alpha * pk_ref[5]
    vals_ref[3, :, :] = alpha
    vals_ref[4, :, :] = alpha * tz
    ints_ref[0, :, :] = iy * W + ix
    ints_ref[1, :, :] = rad_i
    tw = 2 * rad_i + 1
    ints_ref[2, :, :] = jnp.where(maskb, tw * tw, 0)


def _phase1(packed, params, interpret=False):
    grid = _NROWS // _BR
    return pl.pallas_call(
        _phase1_body,
        grid=(grid,),
        in_specs=[
            pl.BlockSpec((1, 128), lambda i: (0, 0)),
            pl.BlockSpec((14, _BR, 128), lambda i: (0, i, 0)),
        ],
        out_specs=[
            pl.BlockSpec((5, _BR, 128), lambda i: (0, i, 0)),
            pl.BlockSpec((3, _BR, 128), lambda i: (0, i, 0)),
        ],
        out_shape=[
            jax.ShapeDtypeStruct((5, _NROWS, 128), jnp.float32),
            jax.ShapeDtypeStruct((3, _NROWS, 128), jnp.int32),
        ],
        interpret=interpret,
    )(params, packed)


_WB = 5000


def _sc_scatter_body(flat_hbm, vals_hbm, out_hbm,
                     idx_v, va, vb, zbuf,
                     acc0, acc1, semi, sema, semb, semz, semw):
    c = lax.axis_index("c")
    s = lax.axis_index("s")
    accs = [acc0, acc1]

    def zfill(i, _):
        zbuf[pl.ds(i * 16, 16)] = jnp.zeros((16,), jnp.float32)
        return 0
    lax.fori_loop(0, _ZB // 16, zfill, 0)
    cpi = pltpu.async_copy(flat_hbm.at[s], idx_v, semi)
    cpa = pltpu.async_copy(
        vals_hbm.at[pl.ds((2 * c + 0) * _NPAD + s * _CHUNK, _CHUNK)], va, sema)
    cpb = pltpu.async_copy(
        vals_hbm.at[pl.ds((2 * c + 1) * _NPAD + s * _CHUNK, _CHUNK)], vb, semb)

    def zero(k):
        for t in range(_STRIPE // _ZB):
            pltpu.async_copy(zbuf, accs[k].at[pl.ds(s * _STRIPE + t * _ZB, _ZB)],
                             semz)

    def zero_wait(k):
        for t in range(_STRIPE // _ZB):
            pltpu.make_async_copy(
                zbuf, accs[k].at[pl.ds(s * _STRIPE + t * _ZB, _ZB)], semz).wait()

    def scatter(k, vv, glo, ghi):
        def group(g, _):
            for b in range(_FIRE):
                j = g * _FIRE + b
                pltpu.async_copy(vv.at[pl.ds(j * 128, 128)],
                                 accs[k].at[idx_v.at[j]], semw, add=True)
            for b in range(_FIRE):
                j = g * _FIRE + b
                pltpu.make_async_copy(vv.at[pl.ds(j * 128, 128)],
                                      accs[k].at[idx_v.at[j]], semw).wait()
            return 0
        lax.fori_loop(glo, ghi, group, 0)

    def writeout(k, chg, ring):
        def buf(t):
            return ring[t % 2].at[pl.ds(0, _WB)]

        def hbm(t):
            return out_hbm.at[pl.ds(chg * _HW + s * _STRIPE + t * _WB, _WB)]
        nt = _STRIPE // _WB
        for t in range(nt):
            if t >= 2:
                pltpu.make_async_copy(buf(t - 2), hbm(t - 2), semw).wait()
            pltpu.sync_copy(accs[k].at[pl.ds(s * _STRIPE + t * _WB, _WB)],
                            buf(t))
            pltpu.async_copy(buf(t), hbm(t), semw)
        for t in (nt - 2, nt - 1):
            pltpu.make_async_copy(buf(t), hbm(t), semw).wait()

    ngrp = _KJ // _FIRE
    zero(0)
    zero(1)
    zero_wait(0)
    zero_wait(1)
    plsc.subcore_barrier()
    cpi.wait()
    cpa.wait()
    scatter(0, va, 0, ngrp)
    cpb.wait()
    scatter(1, vb, 0, ngrp)
    plsc.subcore_barrier()
    writeout(0, 2 * c + 0, [va, vb])
    cpc = pltpu.async_copy(
        vals_hbm.at[pl.ds(4 * _NPAD + s * _CHUNK, _CHUNK)], vb, semb)
    writeout(1, 2 * c + 1, [va, va.at[pl.ds(_WB, _WB)]])
    zero(0)
    zero_wait(0)
    plsc.subcore_barrier()
    cpc.wait()
    half = ngrp // 2
    scatter(0, vb, half * c, half + half * c)
    plsc.subcore_barrier()
    writeout(0, 4 + c, [va, va.at[pl.ds(_WB, _WB)]])


def _sc_scatter(flat3, vals4, interpret=False):
    mesh = plsc.VectorSubcoreMesh(core_axis_name="c", subcore_axis_name="s",
                                  num_cores=2, num_subcores=_NSUB)
    return pl.kernel(
        _sc_scatter_body,
        out_type=jax.ShapeDtypeStruct((6 * _HW,), jnp.float32),
        mesh=mesh,
        scratch_types=[
            pltpu.VMEM((_KJ, 128), jnp.int32),
            pltpu.VMEM((_CHUNK,), jnp.float32),
            pltpu.VMEM((_CHUNK,), jnp.float32),
            pltpu.VMEM((_ZB,), jnp.float32),
            pltpu.VMEM_SHARED((_HW,), jnp.float32),
            pltpu.VMEM_SHARED((_HW,), jnp.float32),
            pltpu.SemaphoreType.DMA,
            pltpu.SemaphoreType.DMA,
            pltpu.SemaphoreType.DMA,
            pltpu.SemaphoreType.DMA,
            pltpu.SemaphoreType.DMA,
        ],
        interpret=interpret,
    )(flat3, vals4)


def _phase3_body(bgp_ref, acc_ref, col_ref, opac_ref, dep_ref):
    acca = acc_ref[3]
    accd = acc_ref[4] + acc_ref[5]
    Tt = jnp.clip(1.0 - acca, 0.0, 1.0)
    col_ref[0, :, :] = acc_ref[0] + Tt * bgp_ref[0, 0]
    col_ref[1, :, :] = acc_ref[1] + Tt * bgp_ref[0, 1]
    col_ref[2, :, :] = acc_ref[2] + Tt * bgp_ref[0, 2]
    opac_ref[:, :] = jnp.clip(acca, 0.0, 1.0)
    dep_ref[:, :] = accd / (acca + 1e-6)


def _phase3(accs, bgp, interpret=False):
    grid = _PROWS // _PBR
    return pl.pallas_call(
        _phase3_body,
        grid=(grid,),
        in_specs=[
            pl.BlockSpec((1, 128), lambda i: (0, 0)),
            pl.BlockSpec((6, _PBR, 128), lambda i: (0, i, 0)),
        ],
        out_specs=[
            pl.BlockSpec((3, _PBR, 128), lambda i: (0, i, 0)),
            pl.BlockSpec((_PBR, 128), lambda i: (i, 0)),
            pl.BlockSpec((_PBR, 128), lambda i: (i, 0)),
        ],
        out_shape=[
            jax.ShapeDtypeStruct((3, _PROWS, 128), jnp.float32),
            jax.ShapeDtypeStruct((_PROWS, 128), jnp.float32),
            jax.ShapeDtypeStruct((_PROWS, 128), jnp.float32),
        ],
        interpret=interpret,
    )(bgp, accs)


def kernel(means3D, means2D, sh, colors_precomp, opacities, scales, rotations,
           theta, rho, viewmatrix, projmatrix, campos, bg):
    pad = _NPAD - _N
    packed = jnp.concatenate(
        [means3D, colors_precomp, opacities, scales, rotations], axis=1)
    packed = jnp.pad(packed, ((0, pad), (0, 0))).T.reshape(14, _NROWS, 128)
    params = jnp.zeros((1, 128), jnp.float32)
    params = params.at[0, :16].set(viewmatrix.reshape(-1))
    params = params.at[0, 16:32].set(projmatrix.reshape(-1))
    params = params.astype(jnp.bfloat16).astype(jnp.float32)
    vals, ints = _phase1(packed, params)
    accs = _sc_scatter(ints[0].reshape(_NSUB, _KJ, 128), vals.reshape(-1))
    bgp = jnp.zeros((1, 128), jnp.float32).at[0, :3].set(bg)
    colf, opacf, depf = _phase3(accs.reshape(6, _PROWS, 128), bgp)
    radnt = ints[1:].reshape(2, _NPAD)
    return (colf.reshape(3, H, W), radnt[0, :_N], depf.reshape(1, H, W),
            opacf.reshape(1, H, W), radnt[1, :_N])


# speedup vs baseline: 1.9485x; 1.7640x over previous
import functools

import jax
import jax.numpy as jnp
from jax import lax
from jax.experimental import pallas as pl
from jax.experimental.pallas import tpu as pltpu
from jax.experimental.pallas import tpu_sc as plsc

H, W = 800, 800
TANFOVX, TANFOVY = 0.5, 0.5
SCALE_MOD = 1.0
_N = 200000
_NPAD = 204800
_NROWS = _NPAD // 128
_BR = 160
_HW = H * W
_PROWS = _HW // 128
_PBR = 1000
_NSUB = 16
_CHUNK = _NPAD // _NSUB
_KJ = _CHUNK // 128
_STRIPE = _HW // _NSUB
_ZB = 8000
_FIRE = 10


def _bfr(v):
    return v.astype(jnp.bfloat16).astype(jnp.float32)


def _phase1_body(params_ref, m3_ref, col_ref, opa_ref, scl_ref, rot_ref,
                 vals_ref, ints_ref):
    def P(i):
        return params_ref[0, i]
    vm = [[P(r * 4 + c) for c in range(4)] for r in range(4)]
    pm = [[P(16 + r * 4 + c) for c in range(4)] for r in range(4)]
    x = _bfr(m3_ref[0])
    y = _bfr(m3_ref[1])
    z = _bfr(m3_ref[2])
    tx = x * vm[0][0] + y * vm[1][0] + z * vm[2][0] + vm[3][0]
    ty = x * vm[0][1] + y * vm[1][1] + z * vm[2][1] + vm[3][1]
    tz = x * vm[0][2] + y * vm[1][2] + z * vm[2][2] + vm[3][2]
    hx = x * pm[0][0] + y * pm[1][0] + z * pm[2][0] + pm[3][0]
    hy = x * pm[0][1] + y * pm[1][1] + z * pm[2][1] + pm[3][1]
    hw = x * pm[0][3] + y * pm[1][3] + z * pm[2][3] + pm[3][3]
    pw = 1.0 / (hw + 1e-7)
    px = ((hx * pw + 1.0) * W - 1.0) * 0.5
    py = ((hy * pw + 1.0) * H - 1.0) * 0.5
    qr = rot_ref[0]
    qx = rot_ref[1]
    qy = rot_ref[2]
    qz = rot_ref[3]
    den = jnp.sqrt(qr * qr + qx * qx + qy * qy + qz * qz) + 1e-8
    r = qr / den
    xq = qx / den
    yq = qy / den
    zq = qz / den
    R = [[1.0 - 2.0 * (yq * yq + zq * zq), 2.0 * (xq * yq - r * zq), 2.0 * (xq * zq + r * yq)],
         [2.0 * (xq * yq + r * zq), 1.0 - 2.0 * (xq * xq + zq * zq), 2.0 * (yq * zq - r * xq)],
         [2.0 * (xq * zq - r * yq), 2.0 * (yq * zq + r * xq), 1.0 - 2.0 * (xq * xq + yq * yq)]]
    s = [scl_ref[0] * SCALE_MOD, scl_ref[1] * SCALE_MOD, scl_ref[2] * SCALE_MOD]
    M = [[_bfr(R[i][k] * s[k]) for k in range(3)] for i in range(3)]
    cov = [[M[i][0] * M[j][0] + M[i][1] * M[j][1] + M[i][2] * M[j][2]
            for j in range(3)] for i in range(3)]
    Wr = [[vm[j][i] for j in range(3)] for i in range(3)]
    covb = [[_bfr(cov[j][k]) for k in range(3)] for j in range(3)]
    T = [[Wr[i][0] * covb[0][k] + Wr[i][1] * covb[1][k] + Wr[i][2] * covb[2][k]
          for k in range(3)] for i in range(3)]
    V = [[_bfr(T[i][0]) * Wr[l][0] + _bfr(T[i][1]) * Wr[l][1] + _bfr(T[i][2]) * Wr[l][2]
          for l in range(3)] for i in range(3)]
    fx = W / (2.0 * TANFOVX)
    fy = H / (2.0 * TANFOVY)
    tzc = jnp.maximum(tz, 1e-3)
    a = fx / tzc
    b = fy / tzc
    c1 = -fx * tx / (tzc * tzc)
    c2 = -fy * ty / (tzc * tzc)
    c00 = a * a * V[0][0] + 2.0 * a * c1 * V[0][2] + c1 * c1 * V[2][2] + 0.3
    c01 = a * b * V[0][1] + a * c2 * V[0][2] + b * c1 * V[1][2] + c1 * c2 * V[2][2]
    c11 = b * b * V[1][1] + 2.0 * b * c2 * V[1][2] + c2 * c2 * V[2][2] + 0.3
    det = c00 * c11 - c01 * c01
    mid = 0.5 * (c00 + c11)
    lam = mid + jnp.sqrt(jnp.maximum(mid * mid - det, 0.1))
    radius_f = jnp.ceil(3.0 * jnp.sqrt(jnp.maximum(lam, 0.0)))
    maskb = ((tz > 0.2) & (px >= 0.0) & (px <= W - 1.0)
             & (py >= 0.0) & (py <= H - 1.0) & (det > 0.0))
    mask = maskb.astype(jnp.float32)
    rad_i = jnp.where(maskb, radius_f, 0.0).astype(jnp.int32)
    ix = jnp.clip(jnp.round(px), 0, W - 1).astype(jnp.int32)
    iy = jnp.clip(jnp.round(py), 0, H - 1).astype(jnp.int32)
    alpha = jnp.clip(opa_ref[0], 0.0, 0.99) * mask
    vals_ref[0, :, :] = alpha * col_ref[0]
    vals_ref[1, :, :] = alpha * col_ref[1]
    vals_ref[2, :, :] = alpha * col_ref[2]
    vals_ref[3, :, :] = alpha
    vals_ref[4, :, :] = alpha * tz
    ints_ref[0, :, :] = iy * W + ix
    ints_ref[1, :, :] = rad_i
    tw = 2 * rad_i + 1
    ints_ref[2, :, :] = jnp.where(maskb, tw * tw, 0)


def _phase1(m3r, colr, opar, sclr, rotr, params, interpret=False):
    grid = _NROWS // _BR
    return pl.pallas_call(
        _phase1_body,
        grid=(grid,),
        in_specs=[
            pl.BlockSpec((1, 128), lambda i: (0, 0)),
            pl.BlockSpec((3, _BR, 128), lambda i: (0, i, 0)),
            pl.BlockSpec((3, _BR, 128), lambda i: (0, i, 0)),
            pl.BlockSpec((1, _BR, 128), lambda i: (0, i, 0)),
            pl.BlockSpec((3, _BR, 128), lambda i: (0, i, 0)),
            pl.BlockSpec((4, _BR, 128), lambda i: (0, i, 0)),
        ],
        out_specs=[
            pl.BlockSpec((5, _BR, 128), lambda i: (0, i, 0)),
            pl.BlockSpec((3, _BR, 128), lambda i: (0, i, 0)),
        ],
        out_shape=[
            jax.ShapeDtypeStruct((5, _NROWS, 128), jnp.float32),
            jax.ShapeDtypeStruct((3, _NROWS, 128), jnp.int32),
        ],
        interpret=interpret,
    )(params, m3r, colr, opar, sclr, rotr)


_WB = 5000


def _sc_scatter_body(flat_hbm, vals_hbm, out_hbm,
                     idx_v, va, vb, zbuf,
                     acc0, acc1, semi, sema, semb, semz, semw):
    c = lax.axis_index("c")
    s = lax.axis_index("s")
    accs = [acc0, acc1]

    def zfill(i, _):
        zbuf[pl.ds(i * 16, 16)] = jnp.zeros((16,), jnp.float32)
        return 0
    lax.fori_loop(0, _ZB // 16, zfill, 0)
    cpi = pltpu.async_copy(flat_hbm.at[s], idx_v, semi)
    cpa = pltpu.async_copy(
        vals_hbm.at[pl.ds((2 * c + 0) * _NPAD + s * _CHUNK, _CHUNK)], va, sema)
    cpb = pltpu.async_copy(
        vals_hbm.at[pl.ds((2 * c + 1) * _NPAD + s * _CHUNK, _CHUNK)], vb, semb)

    def zero(k):
        for t in range(_STRIPE // _ZB):
            pltpu.async_copy(zbuf, accs[k].at[pl.ds(s * _STRIPE + t * _ZB, _ZB)],
                             semz)

    def zero_wait(k):
        for t in range(_STRIPE // _ZB):
            pltpu.make_async_copy(
                zbuf, accs[k].at[pl.ds(s * _STRIPE + t * _ZB, _ZB)], semz).wait()

    def scatter(k, vv, glo, ghi):
        def group(g, _):
            for b in range(_FIRE):
                j = g * _FIRE + b
                pltpu.async_copy(vv.at[pl.ds(j * 128, 128)],
                                 accs[k].at[idx_v.at[j]], semw, add=True)
            for b in range(_FIRE):
                j = g * _FIRE + b
                pltpu.make_async_copy(vv.at[pl.ds(j * 128, 128)],
                                      accs[k].at[idx_v.at[j]], semw).wait()
            return 0
        lax.fori_loop(glo, ghi, group, 0)

    def writeout(k, chg, ring):
        def buf(t):
            return ring[t % 2].at[pl.ds(0, _WB)]

        def hbm(t):
            return out_hbm.at[pl.ds(chg * _HW + s * _STRIPE + t * _WB, _WB)]
        nt = _STRIPE // _WB
        for t in range(nt):
            if t >= 2:
                pltpu.make_async_copy(buf(t - 2), hbm(t - 2), semw).wait()
            pltpu.sync_copy(accs[k].at[pl.ds(s * _STRIPE + t * _WB, _WB)],
                            buf(t))
            pltpu.async_copy(buf(t), hbm(t), semw)
        for t in (nt - 2, nt - 1):
            pltpu.make_async_copy(buf(t), hbm(t), semw).wait()

    ngrp = _KJ // _FIRE
    zero(0)
    zero(1)
    zero_wait(0)
    zero_wait(1)
    plsc.subcore_barrier()
    cpi.wait()
    cpa.wait()
    scatter(0, va, 0, ngrp)
    cpb.wait()
    scatter(1, vb, 0, ngrp)
    plsc.subcore_barrier()
    writeout(0, 2 * c + 0, [va, vb])
    cpc = pltpu.async_copy(
        vals_hbm.at[pl.ds(4 * _NPAD + s * _CHUNK, _CHUNK)], vb, semb)
    writeout(1, 2 * c + 1, [va, va.at[pl.ds(_WB, _WB)]])
    zero(0)
    zero_wait(0)
    plsc.subcore_barrier()
    cpc.wait()
    half = ngrp // 2
    scatter(0, vb, half * c, half + half * c)
    plsc.subcore_barrier()
    writeout(0, 4 + c, [va, va.at[pl.ds(_WB, _WB)]])


def _sc_scatter(flat3, vals4, interpret=False):
    mesh = plsc.VectorSubcoreMesh(core_axis_name="c", subcore_axis_name="s",
                                  num_cores=2, num_subcores=_NSUB)
    return pl.kernel(
        _sc_scatter_body,
        out_type=jax.ShapeDtypeStruct((6 * _HW,), jnp.float32),
        mesh=mesh,
        scratch_types=[
            pltpu.VMEM((_KJ, 128), jnp.int32),
            pltpu.VMEM((_CHUNK,), jnp.float32),
            pltpu.VMEM((_CHUNK,), jnp.float32),
            pltpu.VMEM((_ZB,), jnp.float32),
            pltpu.VMEM_SHARED((_HW,), jnp.float32),
            pltpu.VMEM_SHARED((_HW,), jnp.float32),
            pltpu.SemaphoreType.DMA,
            pltpu.SemaphoreType.DMA,
            pltpu.SemaphoreType.DMA,
            pltpu.SemaphoreType.DMA,
            pltpu.SemaphoreType.DMA,
        ],
        interpret=interpret,
    )(flat3, vals4)


def _phase3_body(bgp_ref, acc_ref, col_ref, opac_ref, dep_ref):
    acca = acc_ref[3]
    accd = acc_ref[4] + acc_ref[5]
    Tt = jnp.clip(1.0 - acca, 0.0, 1.0)
    col_ref[0, :, :] = acc_ref[0] + Tt * bgp_ref[0, 0]
    col_ref[1, :, :] = acc_ref[1] + Tt * bgp_ref[0, 1]
    col_ref[2, :, :] = acc_ref[2] + Tt * bgp_ref[0, 2]
    opac_ref[:, :] = jnp.clip(acca, 0.0, 1.0)
    dep_ref[:, :] = accd / (acca + 1e-6)


def _phase3(accs, bgp, interpret=False):
    grid = _PROWS // _PBR
    return pl.pallas_call(
        _phase3_body,
        grid=(grid,),
        in_specs=[
            pl.BlockSpec((1, 128), lambda i: (0, 0)),
            pl.BlockSpec((6, _PBR, 128), lambda i: (0, i, 0)),
        ],
        out_specs=[
            pl.BlockSpec((3, _PBR, 128), lambda i: (0, i, 0)),
            pl.BlockSpec((_PBR, 128), lambda i: (i, 0)),
            pl.BlockSpec((_PBR, 128), lambda i: (i, 0)),
        ],
        out_shape=[
            jax.ShapeDtypeStruct((3, _PROWS, 128), jnp.float32),
            jax.ShapeDtypeStruct((_PROWS, 128), jnp.float32),
            jax.ShapeDtypeStruct((_PROWS, 128), jnp.float32),
        ],
        interpret=interpret,
    )(bgp, accs)


def kernel(means3D, means2D, sh, colors_precomp, opacities, scales, rotations,
           theta, rho, viewmatrix, projmatrix, campos, bg):
    pad = _NPAD - _N

    def padT(x):
        k = x.shape[1]
        return jnp.pad(x, ((0, pad), (0, 0))).T.reshape(k, _NROWS, 128)
    m3r = padT(means3D)
    colr = padT(colors_precomp)
    opar = padT(opacities)
    sclr = padT(scales)
    rotr = padT(rotations)
    params = jnp.zeros((1, 128), jnp.float32)
    params = params.at[0, :16].set(viewmatrix.reshape(-1))
    params = params.at[0, 16:32].set(projmatrix.reshape(-1))
    params = params.astype(jnp.bfloat16).astype(jnp.float32)
    vals, ints = _phase1(m3r, colr, opar, sclr, rotr, params)
    accs = _sc_scatter(ints[0].reshape(_NSUB, _KJ, 128), vals.reshape(-1))
    bgp = jnp.zeros((1, 128), jnp.float32).at[0, :3].set(bg)
    colf, opacf, depf = _phase3(accs.reshape(6, _PROWS, 128), bgp)
    radnt = ints[1:].reshape(2, _NPAD)
    return (colf.reshape(3, H, W), radnt[0, :_N], depf.reshape(1, H, W),
            opacf.reshape(1, H, W), radnt[1, :_N])


# speedup vs baseline: 1.9735x; 1.0128x over previous
import functools

import jax
import jax.numpy as jnp
from jax import lax
from jax.experimental import pallas as pl
from jax.experimental.pallas import tpu as pltpu
from jax.experimental.pallas import tpu_sc as plsc

H, W = 800, 800
TANFOVX, TANFOVY = 0.5, 0.5
SCALE_MOD = 1.0
_N = 200000
_NPAD = 204800
_NROWS = _NPAD // 128
_BR = 160
_HW = H * W
_PROWS = _HW // 128
_PBR = 1000
_NSUB = 16
_CHUNK = _NPAD // _NSUB
_KJ = _CHUNK // 128
_STRIPE = _HW // _NSUB
_ZB = 8000
_FIRE = 10


def _bfr(v):
    return v.astype(jnp.bfloat16).astype(jnp.float32)


def _phase1_body(params_ref, m3_ref, col_ref, opa_ref, scl_ref, rot_ref,
                 vals_ref, ints_ref):
    def P(i):
        return params_ref[0, i]
    vm = [[P(r * 4 + c) for c in range(4)] for r in range(4)]
    pm = [[P(16 + r * 4 + c) for c in range(4)] for r in range(4)]
    x = _bfr(m3_ref[0])
    y = _bfr(m3_ref[1])
    z = _bfr(m3_ref[2])
    tx = x * vm[0][0] + y * vm[1][0] + z * vm[2][0] + vm[3][0]
    ty = x * vm[0][1] + y * vm[1][1] + z * vm[2][1] + vm[3][1]
    tz = x * vm[0][2] + y * vm[1][2] + z * vm[2][2] + vm[3][2]
    hx = x * pm[0][0] + y * pm[1][0] + z * pm[2][0] + pm[3][0]
    hy = x * pm[0][1] + y * pm[1][1] + z * pm[2][1] + pm[3][1]
    hw = x * pm[0][3] + y * pm[1][3] + z * pm[2][3] + pm[3][3]
    pw = 1.0 / (hw + 1e-7)
    px = ((hx * pw + 1.0) * W - 1.0) * 0.5
    py = ((hy * pw + 1.0) * H - 1.0) * 0.5
    qr = rot_ref[0]
    qx = rot_ref[1]
    qy = rot_ref[2]
    qz = rot_ref[3]
    den = jnp.sqrt(qr * qr + qx * qx + qy * qy + qz * qz) + 1e-8
    r = qr / den
    xq = qx / den
    yq = qy / den
    zq = qz / den
    R = [[1.0 - 2.0 * (yq * yq + zq * zq), 2.0 * (xq * yq - r * zq), 2.0 * (xq * zq + r * yq)],
         [2.0 * (xq * yq + r * zq), 1.0 - 2.0 * (xq * xq + zq * zq), 2.0 * (yq * zq - r * xq)],
         [2.0 * (xq * zq - r * yq), 2.0 * (yq * zq + r * xq), 1.0 - 2.0 * (xq * xq + yq * yq)]]
    s = [scl_ref[0] * SCALE_MOD, scl_ref[1] * SCALE_MOD, scl_ref[2] * SCALE_MOD]
    M = [[_bfr(R[i][k] * s[k]) for k in range(3)] for i in range(3)]
    cov = [[M[i][0] * M[j][0] + M[i][1] * M[j][1] + M[i][2] * M[j][2]
            for j in range(3)] for i in range(3)]
    Wr = [[vm[j][i] for j in range(3)] for i in range(3)]
    covb = [[_bfr(cov[j][k]) for k in range(3)] for j in range(3)]
    T = [[Wr[i][0] * covb[0][k] + Wr[i][1] * covb[1][k] + Wr[i][2] * covb[2][k]
          for k in range(3)] for i in range(3)]
    V = [[_bfr(T[i][0]) * Wr[l][0] + _bfr(T[i][1]) * Wr[l][1] + _bfr(T[i][2]) * Wr[l][2]
          for l in range(3)] for i in range(3)]
    fx = W / (2.0 * TANFOVX)
    fy = H / (2.0 * TANFOVY)
    tzc = jnp.maximum(tz, 1e-3)
    a = fx / tzc
    b = fy / tzc
    c1 = -fx * tx / (tzc * tzc)
    c2 = -fy * ty / (tzc * tzc)
    c00 = a * a * V[0][0] + 2.0 * a * c1 * V[0][2] + c1 * c1 * V[2][2] + 0.3
    c01 = a * b * V[0][1] + a * c2 * V[0][2] + b * c1 * V[1][2] + c1 * c2 * V[2][2]
    c11 = b * b * V[1][1] + 2.0 * b * c2 * V[1][2] + c2 * c2 * V[2][2] + 0.3
    det = c00 * c11 - c01 * c01
    mid = 0.5 * (c00 + c11)
    lam = mid + jnp.sqrt(jnp.maximum(mid * mid - det, 0.1))
    radius_f = jnp.ceil(3.0 * jnp.sqrt(jnp.maximum(lam, 0.0)))
    maskb = ((tz > 0.2) & (px >= 0.0) & (px <= W - 1.0)
             & (py >= 0.0) & (py <= H - 1.0) & (det > 0.0))
    mask = maskb.astype(jnp.float32)
    rad_i = jnp.where(maskb, radius_f, 0.0).astype(jnp.int32)
    ix = jnp.clip(jnp.round(px), 0, W - 1).astype(jnp.int32)
    iy = jnp.clip(jnp.round(py), 0, H - 1).astype(jnp.int32)
    alpha = jnp.clip(opa_ref[0], 0.0, 0.99) * mask
    vals_ref[0, :, :] = alpha * col_ref[0]
    vals_ref[1, :, :] = alpha * col_ref[1]
    vals_ref[2, :, :] = alpha * col_ref[2]
    vals_ref[3, :, :] = alpha
    vals_ref[4, :, :] = alpha * tz
    ints_ref[0, :, :] = iy * W + ix
    ints_ref[1, :, :] = rad_i


def _phase1(m3r, colr, opar, sclr, rotr, params, interpret=False):
    grid = _NROWS // _BR
    return pl.pallas_call(
        _phase1_body,
        grid=(grid,),
        in_specs=[
            pl.BlockSpec((1, 128), lambda i: (0, 0)),
            pl.BlockSpec((3, _BR, 128), lambda i: (0, i, 0)),
            pl.BlockSpec((3, _BR, 128), lambda i: (0, i, 0)),
            pl.BlockSpec((1, _BR, 128), lambda i: (0, i, 0)),
            pl.BlockSpec((3, _BR, 128), lambda i: (0, i, 0)),
            pl.BlockSpec((4, _BR, 128), lambda i: (0, i, 0)),
        ],
        out_specs=[
            pl.BlockSpec((5, _BR, 128), lambda i: (0, i, 0)),
            pl.BlockSpec((2, _BR, 128), lambda i: (0, i, 0)),
        ],
        out_shape=[
            jax.ShapeDtypeStruct((5, _NROWS, 128), jnp.float32),
            jax.ShapeDtypeStruct((2, _NROWS, 128), jnp.int32),
        ],
        interpret=interpret,
    )(params, m3r, colr, opar, sclr, rotr)


_WB = 5000


def _sc_scatter_body(flat_hbm, vals_hbm, out_hbm,
                     idx_v, va, vb, zbuf,
                     acc0, acc1, semi, sema, semb, semz, semw):
    c = lax.axis_index("c")
    s = lax.axis_index("s")
    accs = [acc0, acc1]

    def zfill(i, _):
        zbuf[pl.ds(i * 16, 16)] = jnp.zeros((16,), jnp.float32)
        return 0
    lax.fori_loop(0, _ZB // 16, zfill, 0)
    cpi = pltpu.async_copy(flat_hbm.at[s], idx_v, semi)
    cpa = pltpu.async_copy(
        vals_hbm.at[pl.ds((2 * c + 0) * _NPAD + s * _CHUNK, _CHUNK)], va, sema)
    cpb = pltpu.async_copy(
        vals_hbm.at[pl.ds((2 * c + 1) * _NPAD + s * _CHUNK, _CHUNK)], vb, semb)

    def zero(k):
        for t in range(_STRIPE // _ZB):
            pltpu.async_copy(zbuf, accs[k].at[pl.ds(s * _STRIPE + t * _ZB, _ZB)],
                             semz)

    def zero_wait(k):
        for t in range(_STRIPE // _ZB):
            pltpu.make_async_copy(
                zbuf, accs[k].at[pl.ds(s * _STRIPE + t * _ZB, _ZB)], semz).wait()

    def scatter(k, vv, glo, ghi):
        def group(g, _):
            for b in range(_FIRE):
                j = g * _FIRE + b
                pltpu.async_copy(vv.at[pl.ds(j * 128, 128)],
                                 accs[k].at[idx_v.at[j]], semw, add=True)
            for b in range(_FIRE):
                j = g * _FIRE + b
                pltpu.make_async_copy(vv.at[pl.ds(j * 128, 128)],
                                      accs[k].at[idx_v.at[j]], semw).wait()
            return 0
        lax.fori_loop(glo, ghi, group, 0)

    def writeout(k, chg, ring):
        def buf(t):
            return ring[t % 2].at[pl.ds(0, _WB)]

        def hbm(t):
            return out_hbm.at[pl.ds(chg * _HW + s * _STRIPE + t * _WB, _WB)]
        nt = _STRIPE // _WB
        for t in range(nt):
            if t >= 2:
                pltpu.make_async_copy(buf(t - 2), hbm(t - 2), semw).wait()
            pltpu.sync_copy(accs[k].at[pl.ds(s * _STRIPE + t * _WB, _WB)],
                            buf(t))
            pltpu.async_copy(buf(t), hbm(t), semw)
        for t in (nt - 2, nt - 1):
            pltpu.make_async_copy(buf(t), hbm(t), semw).wait()

    ngrp = _KJ // _FIRE
    zero(0)
    zero(1)
    zero_wait(0)
    zero_wait(1)
    plsc.subcore_barrier()
    cpi.wait()
    cpa.wait()
    scatter(0, va, 0, ngrp)
    cpb.wait()
    scatter(1, vb, 0, ngrp)
    plsc.subcore_barrier()
    writeout(0, 2 * c + 0, [va, vb])
    cpc = pltpu.async_copy(
        vals_hbm.at[pl.ds(4 * _NPAD + s * _CHUNK, _CHUNK)], vb, semb)
    writeout(1, 2 * c + 1, [va, va.at[pl.ds(_WB, _WB)]])
    zero(0)
    zero_wait(0)
    plsc.subcore_barrier()
    cpc.wait()
    half = ngrp // 2
    scatter(0, vb, half * c, half + half * c)
    plsc.subcore_barrier()
    writeout(0, 4 + c, [va, va.at[pl.ds(_WB, _WB)]])


def _sc_scatter(flat3, vals4, interpret=False):
    mesh = plsc.VectorSubcoreMesh(core_axis_name="c", subcore_axis_name="s",
                                  num_cores=2, num_subcores=_NSUB)
    return pl.kernel(
        _sc_scatter_body,
        out_type=jax.ShapeDtypeStruct((6 * _HW,), jnp.float32),
        mesh=mesh,
        scratch_types=[
            pltpu.VMEM((_KJ, 128), jnp.int32),
            pltpu.VMEM((_CHUNK,), jnp.float32),
            pltpu.VMEM((_CHUNK,), jnp.float32),
            pltpu.VMEM((_ZB,), jnp.float32),
            pltpu.VMEM_SHARED((_HW,), jnp.float32),
            pltpu.VMEM_SHARED((_HW,), jnp.float32),
            pltpu.SemaphoreType.DMA,
            pltpu.SemaphoreType.DMA,
            pltpu.SemaphoreType.DMA,
            pltpu.SemaphoreType.DMA,
            pltpu.SemaphoreType.DMA,
        ],
        interpret=interpret,
    )(flat3, vals4)


def _phase3_body(bgp_ref, acc_ref, col_ref, opac_ref, dep_ref):
    acca = acc_ref[3]
    accd = acc_ref[4] + acc_ref[5]
    Tt = jnp.clip(1.0 - acca, 0.0, 1.0)
    col_ref[0, :, :] = acc_ref[0] + Tt * bgp_ref[0, 0]
    col_ref[1, :, :] = acc_ref[1] + Tt * bgp_ref[0, 1]
    col_ref[2, :, :] = acc_ref[2] + Tt * bgp_ref[0, 2]
    opac_ref[:, :] = jnp.clip(acca, 0.0, 1.0)
    dep_ref[:, :] = accd / (acca + 1e-6)


def _phase3(accs, bgp, interpret=False):
    grid = _PROWS // _PBR
    return pl.pallas_call(
        _phase3_body,
        grid=(grid,),
        in_specs=[
            pl.BlockSpec((1, 128), lambda i: (0, 0)),
            pl.BlockSpec((6, _PBR, 128), lambda i: (0, i, 0)),
        ],
        out_specs=[
            pl.BlockSpec((3, _PBR, 128), lambda i: (0, i, 0)),
            pl.BlockSpec((_PBR, 128), lambda i: (i, 0)),
            pl.BlockSpec((_PBR, 128), lambda i: (i, 0)),
        ],
        out_shape=[
            jax.ShapeDtypeStruct((3, _PROWS, 128), jnp.float32),
            jax.ShapeDtypeStruct((_PROWS, 128), jnp.float32),
            jax.ShapeDtypeStruct((_PROWS, 128), jnp.float32),
        ],
        interpret=interpret,
    )(bgp, accs)


def kernel(means3D, means2D, sh, colors_precomp, opacities, scales, rotations,
           theta, rho, viewmatrix, projmatrix, campos, bg):
    pad = _NPAD - _N

    def padT(x):
        k = x.shape[1]
        return jnp.pad(x, ((0, pad), (0, 0))).T.reshape(k, _NROWS, 128)
    m3r = padT(means3D)
    colr = padT(colors_precomp)
    opar = padT(opacities)
    sclr = padT(scales)
    rotr = padT(rotations)
    params = jnp.zeros((1, 128), jnp.float32)
    params = params.at[0, :16].set(viewmatrix.reshape(-1))
    params = params.at[0, 16:32].set(projmatrix.reshape(-1))
    params = params.astype(jnp.bfloat16).astype(jnp.float32)
    vals, ints = _phase1(m3r, colr, opar, sclr, rotr, params)
    accs = _sc_scatter(ints[0].reshape(_NSUB, _KJ, 128), vals.reshape(-1))
    bgp = jnp.zeros((1, 128), jnp.float32).at[0, :3].set(bg)
    colf, opacf, depf = _phase3(accs.reshape(6, _PROWS, 128), bgp)
    rad = ints[1].reshape(_NPAD)[:_N]
    tw = 2 * rad + 1
    nt = jnp.where(rad > 0, tw * tw, 0)
    return (colf.reshape(3, H, W), rad, depf.reshape(1, H, W),
            opacf.reshape(1, H, W), nt)
